# baseline (speedup 1.0000x reference)
import functools
import math

import jax
import jax.numpy as jnp
from jax import lax
from jax.experimental import pallas as pl
from jax.experimental.pallas import tpu as pltpu
from jax.experimental.pallas import tpu_sc as plsc

B, T = 8, 2048
IN_SIZE = 256
HID = 1024
VOCAB = 16384
MAX_SEQ = 4096
FREQ = 256

N_TOK = B * T
NC, NS = 2, 16
NW = NC * NS
TOK_PER_W = N_TOK // NW
CHUNK = 64
N_CHUNKS = TOK_PER_W // CHUNK


def _sc_gather_body(idx_hbm, table_hbm, out_hbm, idx_v, rows_v, sem):
    wid = lax.axis_index("c") * NS + lax.axis_index("s")
    base = wid * TOK_PER_W

    def body(i, carry):
        off = base + i * CHUNK
        pltpu.sync_copy(idx_hbm.at[pl.ds(off, CHUNK)], idx_v)
        pltpu.async_copy(table_hbm.at[idx_v], rows_v, sem).wait()
        pltpu.sync_copy(rows_v, out_hbm.at[pl.ds(off, CHUNK)])
        return carry

    lax.fori_loop(0, N_CHUNKS, body, 0)


def _sc_gather(idx, table):
    mesh = plsc.VectorSubcoreMesh(core_axis_name="c", subcore_axis_name="s")
    f = pl.kernel(
        _sc_gather_body,
        mesh=mesh,
        out_type=jax.ShapeDtypeStruct((N_TOK, HID), jnp.float32),
        scratch_types=[
            pltpu.VMEM((CHUNK,), jnp.int32),
            pltpu.VMEM((CHUNK, HID), jnp.float32),
            pltpu.SemaphoreType.DMA,
        ],
    )
    return f(idx, table)


def _temb_body(targ_ref, w1_ref, b1_ref, w2_ref, b2_ref, o_ref):
    a = targ_ref[...]
    tf = jnp.concatenate([jnp.cos(a), jnp.sin(a)], axis=1)
    h1 = jnp.dot(tf, w1_ref[...], preferred_element_type=jnp.float32,
                 precision=lax.Precision.HIGHEST) + b1_ref[...]
    h1 = h1 * jax.nn.sigmoid(h1)
    o_ref[...] = jnp.dot(h1, w2_ref[...], preferred_element_type=jnp.float32,
                         precision=lax.Precision.HIGHEST) + b2_ref[...]


TOK_BLK = 512
GRID = N_TOK // TOK_BLK
BLK_PER_BATCH = T // TOK_BLK


def _main_body(x_ref, w_ref, b_ref, pid_ref, invf_ref, temb_ref, cond_ref, o_ref):
    h = jnp.dot(x_ref[...], w_ref[...], preferred_element_type=jnp.float32,
                precision=lax.Precision.HIGHEST)
    p = pid_ref[...].astype(jnp.float32)
    args = p * invf_ref[...]
    pos = jnp.concatenate([jnp.sin(args), jnp.cos(args)], axis=1)
    o_ref[...] = h + b_ref[...] + temb_ref[0] + pos + cond_ref[...]


def kernel(x, position_ids, t, condition, token_table, W_in, b_in,
           W_t1, b_t1, W_t2, b_t2):
    xf = x.reshape(N_TOK, IN_SIZE)
    cond_flat = condition.reshape(N_TOK)

    cond_emb = _sc_gather(cond_flat, token_table)

    half_f = FREQ // 2
    tfreqs = jnp.exp(-math.log(10000.0)
                     * jnp.arange(half_f, dtype=jnp.float32) / half_f)
    targs = t[:, None] * tfreqs[None, :]
    temb = pl.pallas_call(
        _temb_body,
        out_shape=jax.ShapeDtypeStruct((B, HID), jnp.float32),
    )(targs, W_t1, b_t1.reshape(1, HID), W_t2, b_t2.reshape(1, HID))

    half_h = HID // 2
    inv_freq = jnp.exp(-(math.log(10000.0) / (half_h - 1))
                       * jnp.arange(half_h, dtype=jnp.float32)).reshape(1, half_h)
    pid2 = position_ids.reshape(N_TOK, 1)

    out = pl.pallas_call(
        _main_body,
        grid=(GRID,),
        in_specs=[
            pl.BlockSpec((TOK_BLK, IN_SIZE), lambda i: (i, 0)),
            pl.BlockSpec((IN_SIZE, HID), lambda i: (0, 0)),
            pl.BlockSpec((1, HID), lambda i: (0, 0)),
            pl.BlockSpec((TOK_BLK, 1), lambda i: (i, 0)),
            pl.BlockSpec((1, half_h), lambda i: (0, 0)),
            pl.BlockSpec((1, 1, HID), lambda i: (i // BLK_PER_BATCH, 0, 0)),
            pl.BlockSpec((TOK_BLK, HID), lambda i: (i, 0)),
        ],
        out_specs=pl.BlockSpec((TOK_BLK, HID), lambda i: (i, 0)),
        out_shape=jax.ShapeDtypeStruct((N_TOK, HID), jnp.float32),
        compiler_params=pltpu.CompilerParams(
            dimension_semantics=("arbitrary",)),
    )(xf, W_in, b_in.reshape(1, HID), pid2, inv_freq,
      temb.reshape(B, 1, HID), cond_emb)

    return out.reshape(B, T, HID)


# speedup vs baseline: 1.0173x; 1.0173x over previous
import functools
import math

import jax
import jax.numpy as jnp
from jax import lax
from jax.experimental import pallas as pl
from jax.experimental.pallas import tpu as pltpu
from jax.experimental.pallas import tpu_sc as plsc

B, T = 8, 2048
IN_SIZE = 256
HID = 1024
VOCAB = 16384
MAX_SEQ = 4096
FREQ = 256

N_TOK = B * T
NC, NS = 2, 16
NW = NC * NS
TOK_PER_W = N_TOK // NW
CHUNK = 32
N_CHUNKS = TOK_PER_W // CHUNK


def _sincos_table():
    half = HID // 2
    scale = math.log(10000.0) / (half - 1)
    freqs = jnp.exp(jnp.arange(half, dtype=jnp.float32) * -scale)
    pos = jnp.arange(MAX_SEQ + 1, dtype=jnp.float32)[:, None] * freqs[None, :]
    emb = jnp.concatenate([jnp.sin(pos), jnp.cos(pos)], axis=1)
    return emb.at[0, :].set(0.0)


def _sc_gather_body(cidx_hbm, pidx_hbm, ttab_hbm, ptab_hbm,
                    cond_out, pos_out,
                    cidx_v, pidx_v, crows_v, prows_v, csem, psem):
    wid = lax.axis_index("c") * NS + lax.axis_index("s")
    base = wid * TOK_PER_W

    def body(i, carry):
        off = base + i * CHUNK
        pltpu.sync_copy(cidx_hbm.at[pl.ds(off, CHUNK)], cidx_v)
        pltpu.sync_copy(pidx_hbm.at[pl.ds(off, CHUNK)], pidx_v)
        cp1 = pltpu.async_copy(ttab_hbm.at[cidx_v], crows_v, csem)
        cp2 = pltpu.async_copy(ptab_hbm.at[pidx_v], prows_v, psem)
        cp1.wait()
        cp2.wait()
        pltpu.sync_copy(crows_v, cond_out.at[pl.ds(off, CHUNK)])
        pltpu.sync_copy(prows_v, pos_out.at[pl.ds(off, CHUNK)])
        return carry

    lax.fori_loop(0, N_CHUNKS, body, 0)


def _sc_gather(cidx, pidx, ttab, ptab):
    mesh = plsc.VectorSubcoreMesh(core_axis_name="c", subcore_axis_name="s")
    f = pl.kernel(
        _sc_gather_body,
        mesh=mesh,
        out_type=(jax.ShapeDtypeStruct((N_TOK, HID), jnp.float32),
                  jax.ShapeDtypeStruct((N_TOK, HID), jnp.float32)),
        scratch_types=[
            pltpu.VMEM((CHUNK,), jnp.int32),
            pltpu.VMEM((CHUNK,), jnp.int32),
            pltpu.VMEM((CHUNK, HID), jnp.float32),
            pltpu.VMEM((CHUNK, HID), jnp.float32),
            pltpu.SemaphoreType.DMA,
            pltpu.SemaphoreType.DMA,
        ],
    )
    return f(cidx, pidx, ttab, ptab)


def _temb_body(targ_ref, w1_ref, b1_ref, w2_ref, b2_ref, o_ref):
    a = targ_ref[...]
    tf = jnp.concatenate([jnp.cos(a), jnp.sin(a)], axis=1)
    h1 = jnp.dot(tf, w1_ref[...], preferred_element_type=jnp.float32,
                 precision=lax.Precision.HIGHEST) + b1_ref[...]
    h1 = h1 * jax.nn.sigmoid(h1)
    o_ref[...] = jnp.dot(h1, w2_ref[...], preferred_element_type=jnp.float32,
                         precision=lax.Precision.HIGHEST) + b2_ref[...]


TOK_BLK = 512
GRID = N_TOK // TOK_BLK
BLK_PER_BATCH = T // TOK_BLK


def _main_body(x_ref, w_ref, b_ref, temb_ref, cond_ref, pos_ref, o_ref):
    h = jnp.dot(x_ref[...], w_ref[...], preferred_element_type=jnp.float32,
                precision=lax.Precision.HIGHEST)
    o_ref[...] = h + b_ref[...] + temb_ref[0] + cond_ref[...] + pos_ref[...]


def kernel(x, position_ids, t, condition, token_table, W_in, b_in,
           W_t1, b_t1, W_t2, b_t2):
    xf = x.reshape(N_TOK, IN_SIZE)
    cond_flat = condition.reshape(N_TOK)
    pos_flat = position_ids.reshape(N_TOK)

    cond_emb, pos_emb = _sc_gather(cond_flat, pos_flat, token_table,
                                   _sincos_table())

    half_f = FREQ // 2
    tfreqs = jnp.exp(-math.log(10000.0)
                     * jnp.arange(half_f, dtype=jnp.float32) / half_f)
    targs = t[:, None] * tfreqs[None, :]
    temb = pl.pallas_call(
        _temb_body,
        out_shape=jax.ShapeDtypeStruct((B, HID), jnp.float32),
    )(targs, W_t1, b_t1.reshape(1, HID), W_t2, b_t2.reshape(1, HID))

    out = pl.pallas_call(
        _main_body,
        grid=(GRID,),
        in_specs=[
            pl.BlockSpec((TOK_BLK, IN_SIZE), lambda i: (i, 0)),
            pl.BlockSpec((IN_SIZE, HID), lambda i: (0, 0)),
            pl.BlockSpec((1, HID), lambda i: (0, 0)),
            pl.BlockSpec((1, 1, HID), lambda i: (i // BLK_PER_BATCH, 0, 0)),
            pl.BlockSpec((TOK_BLK, HID), lambda i: (i, 0)),
            pl.BlockSpec((TOK_BLK, HID), lambda i: (i, 0)),
        ],
        out_specs=pl.BlockSpec((TOK_BLK, HID), lambda i: (i, 0)),
        out_shape=jax.ShapeDtypeStruct((N_TOK, HID), jnp.float32),
        compiler_params=pltpu.CompilerParams(
            dimension_semantics=("arbitrary",)),
    )(xf, W_in, b_in.reshape(1, HID), temb.reshape(B, 1, HID),
      cond_emb, pos_emb)

    return out.reshape(B, T, HID)


# speedup vs baseline: 1.1343x; 1.1150x over previous
import functools
import math

import jax
import jax.numpy as jnp
import numpy as np
from jax import lax
from jax.experimental import pallas as pl
from jax.experimental.pallas import tpu as pltpu
from jax.experimental.pallas import tpu_sc as plsc

B, T = 8, 2048
IN_SIZE = 256
HID = 1024
VOCAB = 16384
MAX_SEQ = 4096
FREQ = 256

N_TOK = B * T
N_PHASE = 2
PH_TOK = N_TOK // N_PHASE
NC, NS = 2, 16
NW = NC * NS
TOK_PER_W = PH_TOK // NW
CHUNK = 32
N_CHUNKS = TOK_PER_W // CHUNK


def _sincos_table_np():
    half = HID // 2
    scale = math.log(10000.0) / (half - 1)
    freqs = np.exp(np.arange(half, dtype=np.float64) * -scale)
    pos = np.arange(MAX_SEQ + 1, dtype=np.float64)[:, None] * freqs[None, :]
    emb = np.concatenate([np.sin(pos), np.cos(pos)], axis=1).astype(np.float32)
    emb[0, :] = 0.0
    return emb


_SINCOS = _sincos_table_np()


def _sc_gather_body(cidx_hbm, pidx_hbm, ttab_hbm, ptab_hbm,
                    cond_out, pos_out,
                    cidx_v, pidx_v, crows_v, prows_v, csem, psem):
    wid = lax.axis_index("c") * NS + lax.axis_index("s")
    base = wid * TOK_PER_W

    def body(i, carry):
        off = base + i * CHUNK
        pltpu.sync_copy(cidx_hbm.at[pl.ds(off, CHUNK)], cidx_v)
        pltpu.sync_copy(pidx_hbm.at[pl.ds(off, CHUNK)], pidx_v)
        cp1 = pltpu.async_copy(ttab_hbm.at[cidx_v], crows_v, csem)
        cp2 = pltpu.async_copy(ptab_hbm.at[pidx_v], prows_v, psem)
        cp1.wait()
        cp2.wait()
        pltpu.sync_copy(crows_v, cond_out.at[pl.ds(off, CHUNK)])
        pltpu.sync_copy(prows_v, pos_out.at[pl.ds(off, CHUNK)])
        return carry

    lax.fori_loop(0, N_CHUNKS, body, 0)


def _sc_gather(cidx, pidx, ttab, ptab):
    mesh = plsc.VectorSubcoreMesh(core_axis_name="c", subcore_axis_name="s")
    f = pl.kernel(
        _sc_gather_body,
        mesh=mesh,
        out_type=(jax.ShapeDtypeStruct((PH_TOK, HID), jnp.float32),
                  jax.ShapeDtypeStruct((PH_TOK, HID), jnp.float32)),
        scratch_types=[
            pltpu.VMEM((CHUNK,), jnp.int32),
            pltpu.VMEM((CHUNK,), jnp.int32),
            pltpu.VMEM((CHUNK, HID), jnp.float32),
            pltpu.VMEM((CHUNK, HID), jnp.float32),
            pltpu.SemaphoreType.DMA,
            pltpu.SemaphoreType.DMA,
        ],
    )
    return f(cidx, pidx, ttab, ptab)


def _temb_body(targ_ref, w1_ref, b1_ref, w2_ref, b2_ref, o_ref):
    a = targ_ref[...]
    tf = jnp.concatenate([jnp.cos(a), jnp.sin(a)], axis=1)
    h1 = jnp.dot(tf, w1_ref[...], preferred_element_type=jnp.float32,
                 precision=lax.Precision.HIGHEST) + b1_ref[...]
    h1 = h1 * jax.nn.sigmoid(h1)
    o_ref[...] = jnp.dot(h1, w2_ref[...], preferred_element_type=jnp.float32,
                         precision=lax.Precision.HIGHEST) + b2_ref[...]


TOK_BLK = 512
PH_GRID = PH_TOK // TOK_BLK
BLK_PER_BATCH = T // TOK_BLK


def _main_body(x_ref, w_ref, b_ref, temb_ref, cond_ref, pos_ref, *rest):
    o_ref = rest[-1]
    h = jnp.dot(x_ref[...], w_ref[...], preferred_element_type=jnp.float32,
                precision=lax.Precision.HIGHEST)
    o_ref[...] = h + b_ref[...] + temb_ref[0] + cond_ref[...] + pos_ref[...]


def _main_phase(phase, xf, W_in, b_in2, temb3, cond_emb, pos_emb, prev_out):
    blk0 = phase * PH_GRID

    def xmap(i):
        return (blk0 + i, 0)

    def tmap(i):
        return ((blk0 + i) // BLK_PER_BATCH, 0, 0)

    in_specs = [
        pl.BlockSpec((TOK_BLK, IN_SIZE), xmap),
        pl.BlockSpec((IN_SIZE, HID), lambda i: (0, 0)),
        pl.BlockSpec((1, HID), lambda i: (0, 0)),
        pl.BlockSpec((1, 1, HID), tmap),
        pl.BlockSpec((TOK_BLK, HID), lambda i: (i, 0)),
        pl.BlockSpec((TOK_BLK, HID), lambda i: (i, 0)),
    ]
    args = [xf, W_in, b_in2, temb3, cond_emb, pos_emb]
    aliases = {}
    if prev_out is not None:
        in_specs.append(pl.BlockSpec(memory_space=pl.ANY))
        args.append(prev_out)
        aliases = {6: 0}
    return pl.pallas_call(
        _main_body,
        grid=(PH_GRID,),
        in_specs=in_specs,
        out_specs=pl.BlockSpec((TOK_BLK, HID), xmap),
        out_shape=jax.ShapeDtypeStruct((N_TOK, HID), jnp.float32),
        input_output_aliases=aliases,
        compiler_params=pltpu.CompilerParams(
            dimension_semantics=("arbitrary",)),
    )(*args)


def kernel(x, position_ids, t, condition, token_table, W_in, b_in,
           W_t1, b_t1, W_t2, b_t2):
    xf = x.reshape(N_TOK, IN_SIZE)
    cond_flat = condition.reshape(N_TOK)
    pos_flat = position_ids.reshape(N_TOK)
    ptab = jnp.asarray(_SINCOS)

    gathered = [
        _sc_gather(
            lax.slice(cond_flat, (p * PH_TOK,), ((p + 1) * PH_TOK,)),
            lax.slice(pos_flat, (p * PH_TOK,), ((p + 1) * PH_TOK,)),
            token_table, ptab)
        for p in range(N_PHASE)
    ]

    half_f = FREQ // 2
    tfreqs = jnp.exp(-math.log(10000.0)
                     * jnp.arange(half_f, dtype=jnp.float32) / half_f)
    targs = t[:, None] * tfreqs[None, :]
    temb = pl.pallas_call(
        _temb_body,
        out_shape=jax.ShapeDtypeStruct((B, HID), jnp.float32),
    )(targs, W_t1, b_t1.reshape(1, HID), W_t2, b_t2.reshape(1, HID))

    b_in2 = b_in.reshape(1, HID)
    temb3 = temb.reshape(B, 1, HID)

    out = None
    for p in range(N_PHASE):
        cond_emb, pos_emb = gathered[p]
        out = _main_phase(p, xf, W_in, b_in2, temb3, cond_emb, pos_emb, out)

    return out.reshape(B, T, HID)
